# baseline (speedup 1.0000x reference)
import functools

import jax
import jax.numpy as jnp
from jax.experimental import pallas as pl
from jax.experimental.pallas import tpu as pltpu

_LOG_2PI = 1.8378770664093453


def _hmm_body(T, B, K, data_ref, init_ref, trans_ref, means_ref, vars_ref,
              alpha_ref, nll_ref, em_ref):
    H = B // 2
    D2 = data_ref.shape[1]
    D = D2 // 2
    N2 = data_ref.shape[0]

    var = vars_ref[...]
    mean = means_ref[...]
    inv_var = 1.0 / var
    Aw = mean * inv_var
    Bw = 0.5 * inv_var
    zKD = jnp.zeros((K, D), dtype=jnp.float32)
    Ablk = jnp.concatenate(
        [jnp.concatenate([Aw, zKD], axis=1),
         jnp.concatenate([zKD, Aw], axis=1)], axis=0)
    Bblk = jnp.concatenate(
        [jnp.concatenate([Bw, zKD], axis=1),
         jnp.concatenate([zKD, Bw], axis=1)], axis=0)
    M = 0.5 * (jnp.log(var) + mean * mean * inv_var)
    ones_row = jnp.ones((1, D), dtype=jnp.float32)
    c_row = -0.5 * D * _LOG_2PI - jax.lax.dot_general(
        ones_row, M, (((1,), (1,)), ((), ())),
        preferred_element_type=jnp.float32)
    c_ext = jnp.concatenate([c_row, c_row], axis=1)

    CH = 4096
    for i in range(N2 // CH):
        x = data_ref[pl.ds(i * CH, CH), :]
        lp = (jax.lax.dot_general(x, Ablk, (((1,), (1,)), ((), ())),
                                  preferred_element_type=jnp.float32)
              - jax.lax.dot_general(x * x, Bblk, (((1,), (1,)), ((), ())),
                                    preferred_element_type=jnp.float32)
              + c_ext)
        em_ref[pl.ds(i * CH, CH), :] = jnp.exp(lp)

    P = trans_ref[...]
    zKK = jnp.zeros((K, K), dtype=jnp.float32)
    Pblk = jnp.concatenate(
        [jnp.concatenate([P, zKK], axis=1),
         jnp.concatenate([zKK, P], axis=1)], axis=0)
    ip = init_ref[...]
    ip2 = jnp.concatenate([ip, ip], axis=1)
    u = ip2 * em_ref[0:H, :]
    la1 = jnp.zeros((H, 1), dtype=jnp.float32)
    la2 = jnp.zeros((H, 1), dtype=jnp.float32)

    def step(t, carry):
        u, la1, la2 = carry
        r1 = jnp.sum(u[:, :K], axis=1, keepdims=True)
        r2 = jnp.sum(u[:, K:], axis=1, keepdims=True)
        rc1 = jnp.maximum(r1, 1.2e-38)
        rc2 = jnp.maximum(r2, 1.2e-38)
        em_t = em_ref[pl.ds(pl.multiple_of(t * H, H), H), :]
        s = jnp.concatenate([em_t[:, :K] / rc1, em_t[:, K:] / rc2], axis=1)
        m = jax.lax.dot_general(u, Pblk, (((1,), (0,)), ((), ())),
                                preferred_element_type=jnp.float32)
        return (m * s, la1 + jnp.log(rc1), la2 + jnp.log(rc2))

    u, la1, la2 = jax.lax.fori_loop(1, T, step, (u, la1, la2), unroll=16)

    rT1 = jnp.sum(u[:, :K], axis=1, keepdims=True)
    rT2 = jnp.sum(u[:, K:], axis=1, keepdims=True)
    alpha_ref[0:H, :] = u[:, :K] / rT1
    alpha_ref[H:2 * H, :] = u[:, K:] / rT2
    total = (jnp.sum(la1) + jnp.sum(la2)
             + jnp.sum(jnp.log(rT1)) + jnp.sum(jnp.log(rT2)))
    nll_ref[...] = jnp.full((1, 1), -total, dtype=jnp.float32)


def kernel(data, batch_sizes, initial_probs, transition_probs, means,
           variances):
    T = batch_sizes.shape[0]
    N = data.shape[0]
    B = N // T
    K = transition_probs.shape[0]

    D = data.shape[1]
    data_packed = (data.reshape(T, 2, B // 2, D)
                   .transpose(0, 2, 1, 3)
                   .reshape(T * (B // 2), 2 * D))

    body = functools.partial(_hmm_body, T, B, K)
    alpha, nll = pl.pallas_call(
        body,
        out_shape=[
            jax.ShapeDtypeStruct((B, K), jnp.float32),
            jax.ShapeDtypeStruct((1, 1), jnp.float32),
        ],
        scratch_shapes=[pltpu.VMEM((T * (B // 2), 2 * K), jnp.float32)],
    )(data_packed, initial_probs.reshape(1, K), transition_probs, means,
      variances)
    return alpha, nll.reshape(1)
